# baseline (speedup 1.0000x reference)
import functools

import jax
import jax.numpy as jnp
from jax import lax
from jax.experimental import pallas as pl
from jax.experimental.pallas import tpu as pltpu
from jax.experimental.pallas import tpu_sc as plsc

_NC = 2
_NS = 16
_NW = _NC * _NS


def _prep_body(row_ref, adj_ref, *, seq_len, tpw, rows_per_batch):
    b = pl.program_id(0)
    row = row_ref[0]
    nv = row.shape[1]
    col = jnp.reshape(row, (nv, 1))
    eq = col == row
    ii = lax.broadcasted_iota(jnp.int32, (nv, nv), 0)
    jj = lax.broadcasted_iota(jnp.int32, (nv, nv), 1)
    dup_later = jnp.any(eq & (ii > jj), axis=0, keepdims=True)
    keep = (row != 0) & jnp.logical_not(dup_later)
    adj = jnp.where(keep, row + b * seq_len, jnp.int32(0x7FFFFF00))
    for k in range(rows_per_batch):
        w = nv // rows_per_batch
        adj_ref[k : k + 1, :] = adj[:, k * w : (k + 1) * w]


@functools.partial(jax.jit, static_argnames=("seq_len", "tpw"))
def _prep(vision_indices, *, seq_len, tpw):
    b, nv = vision_indices.shape
    rpb = nv // tpw
    row3 = vision_indices.reshape(b, 1, nv)
    return pl.pallas_call(
        functools.partial(
            _prep_body, seq_len=seq_len, tpw=tpw, rows_per_batch=rpb
        ),
        grid=(b,),
        in_specs=[
            pl.BlockSpec((1, 1, nv), lambda i: (i, 0, 0)),
        ],
        out_specs=pl.BlockSpec((rpb, tpw), lambda i: (i, 0)),
        out_shape=jax.ShapeDtypeStruct((b * rpb, tpw), jnp.int32),
    )(row3)


def _make_scatter(tok, hdim, cw, nbuf, trash_stride):
    tpw = tok // _NW
    nch = tpw // cw
    mesh = plsc.VectorSubcoreMesh(
        core_axis_name="c",
        subcore_axis_name="s",
        num_cores=_NC,
        num_subcores=_NS,
    )

    @functools.partial(
        pl.kernel,
        mesh=mesh,
        out_type=(),
        scratch_types=[
            pltpu.VMEM((nch, cw), jnp.int32),
            pltpu.VMEM((nch, cw), jnp.int32),
            pltpu.VMEM((nbuf, cw, hdim), jnp.float32),
            pltpu.VMEM((1, hdim), jnp.float32),
            [pltpu.SemaphoreType.DMA for _ in range(nbuf)],
            [pltpu.SemaphoreType.DMA for _ in range(nbuf)],
        ],
    )
    def scatter_k(vision_hbm, adj_hbm, src_hbm, text_hbm, out_ref, idx_v,
                  src_v, rows_v, row_v, sems_l, sems_s):
        c = lax.axis_index("c")
        s = lax.axis_index("s")
        wid = s * _NC + c
        pltpu.sync_copy(adj_hbm.at[wid], idx_v)
        pltpu.sync_copy(src_hbm.at[wid], src_v)
        loadd = [None] * nch
        scatd = [None] * nch

        def start_load(ch):
            b = ch % nbuf
            if ch >= nbuf:
                scatd[ch - nbuf].wait()
            loadd[ch] = pltpu.async_copy(
                vision_hbm.at[src_v.at[ch]],
                rows_v.at[b],
                sems_l[b],
            )

        def issue_scatter(ch):
            b = ch % nbuf
            loadd[ch].wait()
            scatd[ch] = pltpu.async_copy(
                rows_v.at[b], out_ref.at[idx_v.at[ch]], sems_s[b]
            )

        for ch in range(nch):
            start_load(ch)
            if ch >= nbuf - 1:
                issue_scatter(ch - (nbuf - 1))
        for ch in range(max(0, nch - (nbuf - 1)), nch):
            issue_scatter(ch)
        for ch in range(max(0, nch - nbuf), nch):
            scatd[ch].wait()

        plsc.subcore_barrier()

        @pl.when(s == 0)
        def _():
            trash0 = pl.multiple_of(c * trash_stride, 8)
            pltpu.sync_copy(text_hbm.at[pl.ds(trash0, 1)], row_v)
            pltpu.sync_copy(row_v, out_ref.at[pl.ds(trash0, 1)])

    return scatter_k


def kernel(vision_embeddings, text_embeddings, vision_indices):
    b, s, h = text_embeddings.shape
    tok = vision_embeddings.shape[0]
    tpw = tok // _NW
    cw = 8
    nbuf = 7
    assert tok % _NW == 0 and tpw % cw == 0

    n = b * s
    flat_text = text_embeddings.reshape(n, h)
    adj = _prep(vision_indices.astype(jnp.int32), seq_len=s, tpw=tpw)

    flat_adj = adj.reshape(-1)
    iota = lax.iota(jnp.int32, tok)
    tgt_s, src_s = lax.sort((flat_adj, iota), num_keys=1)
    rank_w = iota // tpw
    trash = (rank_w % _NC) * s
    tgt_f = jnp.where(tgt_s >= jnp.int32(n), trash, tgt_s)
    adj3 = tgt_f.reshape(_NW, tpw // cw, cw)
    src3 = src_s.astype(jnp.int32).reshape(_NW, tpw // cw, cw)

    scatter_k = _make_scatter(tok, h, cw, nbuf, s)
    out_ref = jax.new_ref(flat_text)
    scatter_k(vision_embeddings, adj3, src3, flat_text, out_ref)
    return out_ref[...].reshape(b, s, h)


# speedup vs baseline: 1.0429x; 1.0429x over previous
import functools

import jax
import jax.numpy as jnp
from jax import lax
from jax.experimental import pallas as pl
from jax.experimental.pallas import tpu as pltpu
from jax.experimental.pallas import tpu_sc as plsc

_NC = 2
_NS = 16
_NW = _NC * _NS


def _prep_body(row_ref, adj_ref, *, seq_len, tpw, rows_per_batch):
    b = pl.program_id(0)
    row = row_ref[0]
    nv = row.shape[1]
    col = jnp.reshape(row, (nv, 1))
    eq = col == row
    ii = lax.broadcasted_iota(jnp.int32, (nv, nv), 0)
    jj = lax.broadcasted_iota(jnp.int32, (nv, nv), 1)
    dup_later = jnp.any(eq & (ii > jj), axis=0, keepdims=True)
    keep = (row != 0) & jnp.logical_not(dup_later)
    jl = lax.broadcasted_iota(jnp.int32, (1, nv), 1)
    trash = (((b * nv + jl) // tpw) % _NC) * seq_len
    adj = jnp.where(keep, row + b * seq_len, trash)
    for k in range(rows_per_batch):
        w = nv // rows_per_batch
        adj_ref[k : k + 1, :] = adj[:, k * w : (k + 1) * w]


@functools.partial(jax.jit, static_argnames=("seq_len", "tpw"))
def _prep(vision_indices, *, seq_len, tpw):
    b, nv = vision_indices.shape
    rpb = nv // tpw
    row3 = vision_indices.reshape(b, 1, nv)
    return pl.pallas_call(
        functools.partial(
            _prep_body, seq_len=seq_len, tpw=tpw, rows_per_batch=rpb
        ),
        grid=(b,),
        in_specs=[
            pl.BlockSpec((1, 1, nv), lambda i: (i, 0, 0)),
        ],
        out_specs=pl.BlockSpec((rpb, tpw), lambda i: (i, 0)),
        out_shape=jax.ShapeDtypeStruct((b * rpb, tpw), jnp.int32),
    )(row3)


def _make_scatter(tok, hdim, cw, nbuf, trash_stride):
    tpw = tok // _NW
    nch = tpw // cw
    mesh = plsc.VectorSubcoreMesh(
        core_axis_name="c",
        subcore_axis_name="s",
        num_cores=_NC,
        num_subcores=_NS,
    )

    @functools.partial(
        pl.kernel,
        mesh=mesh,
        out_type=(),
        scratch_types=[
            pltpu.VMEM((nch, cw), jnp.int32),
            pltpu.VMEM((nbuf, cw, hdim), jnp.float32),
            pltpu.VMEM((1, hdim), jnp.float32),
            [pltpu.SemaphoreType.DMA for _ in range(nbuf)],
            [pltpu.SemaphoreType.DMA for _ in range(nbuf)],
        ],
    )
    def scatter_k(vision_hbm, adj_hbm, text_hbm, out_ref, idx_v, rows_v,
                  row_v, sems_l, sems_s):
        c = lax.axis_index("c")
        s = lax.axis_index("s")
        wid = s * _NC + c
        pltpu.sync_copy(adj_hbm.at[wid], idx_v)
        loadd = [None] * nch
        scatd = [None] * nch

        def start_load(ch):
            b = ch % nbuf
            if ch >= nbuf:
                scatd[ch - nbuf].wait()
            loadd[ch] = pltpu.async_copy(
                vision_hbm.at[pl.ds(wid * tpw + ch * cw, cw)],
                rows_v.at[b],
                sems_l[b],
            )

        def issue_scatter(ch):
            b = ch % nbuf
            loadd[ch].wait()
            scatd[ch] = pltpu.async_copy(
                rows_v.at[b], out_ref.at[idx_v.at[ch]], sems_s[b]
            )

        for ch in range(nch):
            start_load(ch)
            if ch >= nbuf - 1:
                issue_scatter(ch - (nbuf - 1))
        for ch in range(max(0, nch - (nbuf - 1)), nch):
            issue_scatter(ch)
        for ch in range(max(0, nch - nbuf), nch):
            scatd[ch].wait()

        plsc.subcore_barrier()

        @pl.when(s == 0)
        def _():
            trash0 = pl.multiple_of(c * trash_stride, 8)
            pltpu.sync_copy(text_hbm.at[pl.ds(trash0, 1)], row_v)
            pltpu.sync_copy(row_v, out_ref.at[pl.ds(trash0, 1)])

    return scatter_k


def kernel(vision_embeddings, text_embeddings, vision_indices):
    b, s, h = text_embeddings.shape
    tok = vision_embeddings.shape[0]
    tpw = tok // _NW
    cw = 8
    nbuf = 7
    assert tok % _NW == 0 and tpw % cw == 0

    flat_text = text_embeddings.reshape(b * s, h)
    adj = _prep(vision_indices.astype(jnp.int32), seq_len=s, tpw=tpw)
    adj3 = adj.reshape(_NW, tpw // cw, cw)

    scatter_k = _make_scatter(tok, h, cw, nbuf, s)
    out_ref = jax.new_ref(flat_text)
    scatter_k(vision_embeddings, adj3, flat_text, out_ref)
    return out_ref[...].reshape(b, s, h)
